# baseline (speedup 1.0000x reference)
import functools

import jax
import jax.numpy as jnp
from jax import lax
from jax.experimental import pallas as pl
from jax.experimental.pallas import tpu as pltpu
from jax.experimental.pallas import tpu_sc as plsc

_INVALID = 1 << 20


def _make_router(N, HP, H, E, B, TT, EP):
    NT = N // TT
    tiles_per_batch = NT // B

    def body(xc_ref, aq_ref, w1c_ref, w1q_ref, w2_ref, b2_ref,
             w_ref, r_ref, cnt_ref, carry_ref):
        t = pl.program_id(0)
        bidx = t // tiles_per_batch
        xc = xc_ref[...]
        hdn = jnp.dot(xc, w1c_ref[...], preferred_element_type=jnp.float32)
        abias = jnp.dot(aq_ref[...], w1q_ref[...], preferred_element_type=jnp.float32)
        rows = jax.lax.broadcasted_iota(jnp.int32, abias.shape, 0)
        brow = jnp.sum(jnp.where(rows == bidx, abias, 0.0), axis=0, keepdims=True)
        hdn = jnp.tanh(hdn + brow)
        logits = jnp.dot(hdn, w2_ref[...], preferred_element_type=jnp.float32)
        logits = logits + b2_ref[0:1, :]
        col = jax.lax.broadcasted_iota(jnp.int32, logits.shape, 1)
        valid = col < E
        logits = jnp.where(valid, logits, -jnp.inf)
        m = jnp.max(logits, axis=1, keepdims=True)
        ex = jnp.where(valid, jnp.exp(logits - m), 0.0)
        gate = ex / jnp.sum(ex, axis=1, keepdims=True)
        m1 = jnp.max(gate, axis=1, keepdims=True)
        i1 = jnp.min(jnp.where(gate == m1, col, EP + 1), axis=1, keepdims=True)
        sel1 = col == i1
        g2 = jnp.where(sel1 | ~valid, -1.0, gate)
        m2 = jnp.max(g2, axis=1, keepdims=True)
        i2 = jnp.min(jnp.where(g2 == m2, col, EP + 1), axis=1, keepdims=True)
        sel = sel1 | (col == i2)
        w_ref[...] = jnp.where(sel, gate, 0.0)

        onehot = sel.astype(jnp.int32)
        incl = onehot
        sh = 1
        while sh < TT:
            incl = incl + jnp.concatenate(
                [jnp.zeros((sh, EP), jnp.int32), incl[:-sh]], axis=0)
            sh *= 2
        excl = incl - onehot

        @pl.when(t == 0)
        def _init():
            carry_ref[...] = jnp.zeros_like(carry_ref)

        carry = carry_ref[0:1, :]
        r_ref[...] = jnp.where(sel, carry + excl, _INVALID)
        newc = carry + incl[TT - 1:TT, :]
        carry_ref[...] = jnp.broadcast_to(newc, carry_ref.shape)
        cnt_ref[...] = jnp.broadcast_to(newc, cnt_ref.shape)

    return body


def _make_dispatch_gather(N, E, T, Pmax, NW, NC, H):
    ENT = N * E // NW
    NCH = ENT // 16
    NV = ENT // 4
    NVR = NV // 128
    TPT = N // NW
    CX = 32
    NCX = TPT // CX
    mesh = plsc.VectorSubcoreMesh(core_axis_name="c", subcore_axis_name="s")

    @functools.partial(
        pl.kernel, mesh=mesh,
        compiler_params=pltpu.CompilerParams(needs_layout_passes=False),
        out_type=[
            jax.ShapeDtypeStruct((Pmax + 16,), jnp.float32),
            jax.ShapeDtypeStruct((2 * N + 16,), jnp.int32),
            jax.ShapeDtypeStruct((Pmax, H), jnp.float32),
        ],
        scratch_types=[
            pltpu.VMEM((ENT,), jnp.int32),
            pltpu.VMEM((ENT,), jnp.float32),
            pltpu.VMEM((16,), jnp.int32),
            pltpu.VMEM((16,), jnp.int32),
            pltpu.VMEM((NVR, 128), jnp.int32),
            pltpu.VMEM((NVR, 128), jnp.int32),
            pltpu.VMEM((NVR, 128), jnp.float32),
            pltpu.VMEM((NCX, CX), jnp.int32),
            pltpu.VMEM((NCX, CX), jnp.int32),
            pltpu.VMEM((2, CX, H), jnp.float32),
            pltpu.SemaphoreType.DMA,
            pltpu.SemaphoreType.DMA,
            pltpu.SemaphoreType.DMA,
            pltpu.SemaphoreType.DMA,
            pltpu.SemaphoreType.DMA,
        ],
    )
    def k(rflat, wflat, cnt_hbm, x_hbm, ws_hbm, posab_hbm, xg_hbm,
          rv, wv, cv, basev, posl, slotl, wvl, pA, pB, xbuf,
          sem, sl0, sl1, ss0, ss1):
        wid = lax.axis_index("s") * NC + lax.axis_index("c")
        base_ent = wid * ENT
        tok0 = wid * TPT
        pltpu.sync_copy(rflat.at[pl.ds(base_ent, ENT)], rv)
        pltpu.sync_copy(wflat.at[pl.ds(base_ent, ENT)], wv)
        pltpu.sync_copy(cnt_hbm, cv)
        cvec = cv[...]
        pc = ((cvec + (T - 1)) // T) * T
        incl = plsc.cumsum(pc)
        basev[...] = incl - pc
        lane = lax.iota(jnp.int32, 16)
        e16 = jnp.bitwise_and(lane, E - 1)

        def chunk(ci, acc):
            off = ci * 16
            r16 = rv[pl.ds(off, 16)]
            w16 = wv[pl.ds(off, 16)]
            gidx = base_ent + off + lane
            i16 = gidx // E
            be = plsc.load_gather(basev, [e16])
            pos = be + r16
            isok = r16 < _INVALID
            vi = jnp.where(isok, 1, 0).astype(jnp.int32)
            grp0 = jnp.where(lane < 8, vi, 0)
            grp1 = jnp.where(lane >= 8, vi, 0)
            rank = jnp.where(lane < 8, plsc.cumsum(grp0), plsc.cumsum(grp1))
            slot = 2 * i16 + rank - 1
            cidx = ci * 4 + jnp.where(lane < 8, 0, 2) + rank - 1
            crow = cidx >> 7
            ccol = jnp.bitwise_and(cidx, 127)
            plsc.store_scatter(posl, [crow, ccol], pos, mask=isok)
            plsc.store_scatter(slotl, [crow, ccol], slot, mask=isok)
            plsc.store_scatter(wvl, [crow, ccol], w16, mask=isok)
            lt = i16 - tok0
            xrow = lt >> 5
            xcol = jnp.bitwise_and(lt, CX - 1)
            plsc.store_scatter(pA, [xrow, xcol], pos, mask=isok & (rank == 1))
            plsc.store_scatter(pB, [xrow, xcol], pos, mask=isok & (rank == 2))
            return acc

        lax.fori_loop(0, NCH, chunk, 0)

        copies = []
        for j in range(NVR):
            copies.append(pltpu.async_copy(wvl.at[j], ws_hbm.at[posl.at[j]], sem))
            copies.append(pltpu.async_copy(posl.at[j], posab_hbm.at[slotl.at[j]], sem))

        lsem = [sl0, sl1]
        ssem = [ss0, ss1]
        loads = [None] * NCX
        prev_sc = [None, None]
        loads[0] = pltpu.async_copy(
            x_hbm.at[pl.ds(tok0, CX)], xbuf.at[0], lsem[0])
        for cx in range(NCX):
            b = cx & 1
            if cx + 1 < NCX:
                b2 = (cx + 1) & 1
                if prev_sc[b2] is not None:
                    for c in prev_sc[b2]:
                        c.wait()
                    prev_sc[b2] = None
                loads[cx + 1] = pltpu.async_copy(
                    x_hbm.at[pl.ds(tok0 + (cx + 1) * CX, CX)],
                    xbuf.at[b2], lsem[b2])
            loads[cx].wait()
            prev_sc[b] = [
                pltpu.async_copy(xbuf.at[b], xg_hbm.at[pA.at[cx]], ssem[b]),
                pltpu.async_copy(xbuf.at[b], xg_hbm.at[pB.at[cx]], ssem[b]),
            ]
        for pend in prev_sc:
            if pend is not None:
                for c in pend:
                    c.wait()
        for c in copies:
            c.wait()

    return k


def _make_sparse_experts():
    def body(texp_ref, xg_ref, w_ref, wa_ref, ba_ref, wb_ref, bb_ref, yg_ref):
        xb = xg_ref[...].astype(jnp.bfloat16)
        h1 = jnp.dot(xb, wa_ref[0], preferred_element_type=jnp.float32)
        h1 = jnp.maximum(h1 + ba_ref[0], 0.0).astype(jnp.bfloat16)
        y = jnp.dot(h1, wb_ref[0], preferred_element_type=jnp.float32)
        yg_ref[...] = (y + bb_ref[0]) * w_ref[:, 0:1]

    return body


def _make_combine(N, H, NW, NC, CT):
    TPT = N // NW
    mesh = plsc.VectorSubcoreMesh(core_axis_name="c", subcore_axis_name="s")

    NCH6 = TPT // CT

    @functools.partial(
        pl.kernel, mesh=mesh,
        out_type=jax.ShapeDtypeStruct((N, H), jnp.float32),
        scratch_types=[
            pltpu.VMEM((2 * TPT,), jnp.int32),
            pltpu.VMEM((2, 2 * CT, H), jnp.float32),
            pltpu.VMEM((2, CT, H), jnp.float32),
            pltpu.SemaphoreType.DMA,
            pltpu.SemaphoreType.DMA,
            pltpu.SemaphoreType.DMA,
            pltpu.SemaphoreType.DMA,
        ],
    )
    def k(yg_hbm, posab_hbm, out_hbm, idxv, rows, obuf, g0, g1, w0, w1):
        wid = lax.axis_index("s") * NC + lax.axis_index("c")
        base = wid * TPT
        HC = H // 16
        pltpu.sync_copy(posab_hbm.at[pl.ds(2 * base, 2 * TPT)], idxv)
        gsem = [g0, g1]
        wsem = [w0, w1]
        gat = [None] * NCH6
        prev_wb = [None, None]
        gat[0] = pltpu.async_copy(
            yg_hbm.at[idxv.at[pl.ds(0, 2 * CT)]], rows.at[0], gsem[0])
        for ci in range(NCH6):
            b = ci & 1
            if ci + 1 < NCH6:
                gat[ci + 1] = pltpu.async_copy(
                    yg_hbm.at[idxv.at[pl.ds(2 * (ci + 1) * CT, 2 * CT)]],
                    rows.at[(ci + 1) & 1], gsem[(ci + 1) & 1])
            gat[ci].wait()
            if prev_wb[b] is not None:
                prev_wb[b].wait()
                prev_wb[b] = None

            def add(q, a2):
                j = q // HC
                s = (q % HC) * 16
                obuf[b, j, pl.ds(s, 16)] = (
                    rows[b, 2 * j, pl.ds(s, 16)] + rows[b, 2 * j + 1, pl.ds(s, 16)])
                return a2

            lax.fori_loop(0, CT * HC, add, 0)
            prev_wb[b] = pltpu.async_copy(
                obuf.at[b], out_hbm.at[pl.ds(base + ci * CT, CT)], wsem[b])
        for pend in prev_wb:
            if pend is not None:
                pend.wait()

    return k


def kernel(token_x, aspect_q, token_score, W1, b1, W2, b2, Wa, ba, Wb, bb):
    B, M, H = token_x.shape
    E = Wa.shape[0]
    HID = Wa.shape[2]
    N = B * M
    TT = 256
    EP = 128
    T = 256
    NT = N // TT
    HP = H + 128
    Pmax = 2 * N + E * T
    NTS = Pmax // T
    NTT = ((NTS + 15) // 16) * 16
    NW, NC = 32, 2
    CG = 64
    CT = 16

    X = token_x.reshape(N, H)
    ts = token_score.reshape(N, 1)
    Xc = jnp.concatenate([X, jnp.broadcast_to(ts, (N, 128))], axis=1)
    W1c = jnp.concatenate(
        [W1[:H], W1[2 * H][None, :], jnp.zeros((127, H), jnp.float32)], axis=0)
    aqp = jnp.zeros((8, HP), jnp.float32)
    aqp = aqp.at[:B, :H].set(aspect_q)
    aqp = aqp.at[:, H].set(1.0)
    W1q = jnp.concatenate(
        [W1[H:2 * H], b1[None, :], jnp.zeros((127, H), jnp.float32)], axis=0)
    W2p = jnp.pad(W2, ((0, 0), (0, EP - E)))
    b2p = jnp.broadcast_to(jnp.pad(b2, (0, EP - E))[None, :], (8, EP))

    w_full, r_full, cnts = pl.pallas_call(
        _make_router(N, HP, H, E, B, TT, EP),
        grid=(NT,),
        in_specs=[
            pl.BlockSpec((TT, HP), lambda t: (t, 0)),
            pl.BlockSpec((8, HP), lambda t: (0, 0)),
            pl.BlockSpec((HP, H), lambda t: (0, 0)),
            pl.BlockSpec((HP, H), lambda t: (0, 0)),
            pl.BlockSpec((H, EP), lambda t: (0, 0)),
            pl.BlockSpec((8, EP), lambda t: (0, 0)),
        ],
        out_specs=[
            pl.BlockSpec((TT, EP), lambda t: (t, 0)),
            pl.BlockSpec((TT, EP), lambda t: (t, 0)),
            pl.BlockSpec((8, EP), lambda t: (0, 0)),
        ],
        out_shape=[
            jax.ShapeDtypeStruct((N, EP), jnp.float32),
            jax.ShapeDtypeStruct((N, EP), jnp.int32),
            jax.ShapeDtypeStruct((8, EP), jnp.int32),
        ],
        scratch_shapes=[pltpu.VMEM((8, EP), jnp.int32)],
    )(Xc, aqp, W1c, W1q, W2p, b2p)

    rflat = r_full[:, :E].reshape(-1)
    wflat = w_full[:, :E].reshape(-1)
    cnt16 = jnp.zeros((16,), jnp.int32).at[:E].set(cnts[0, :E])

    ws, posab, Xg = _make_dispatch_gather(N, E, T, Pmax, NW, NC, H)(
        rflat, wflat, cnt16, X)

    ends = jnp.cumsum(((cnts[0, :E] + T - 1) // T) * T)
    tile_base = jnp.arange(NTS, dtype=jnp.int32) * T
    texp = jnp.minimum(
        jnp.sum((tile_base[:, None] >= ends[None, :]).astype(jnp.int32), axis=1),
        E - 1)

    wbcast = jnp.broadcast_to(ws[:Pmax, None], (Pmax, EP))
    Wab = Wa.astype(jnp.bfloat16)
    Wbb = Wb.astype(jnp.bfloat16)
    ba3 = ba.reshape(E, 1, HID)
    bb3 = bb.reshape(E, 1, H)
    grid_spec = pltpu.PrefetchScalarGridSpec(
        num_scalar_prefetch=1,
        grid=(NTS,),
        in_specs=[
            pl.BlockSpec((T, H), lambda t, texp_ref: (t, 0)),
            pl.BlockSpec((T, EP), lambda t, texp_ref: (t, 0)),
            pl.BlockSpec((1, H, HID), lambda t, texp_ref: (texp_ref[t], 0, 0)),
            pl.BlockSpec((1, 1, HID), lambda t, texp_ref: (texp_ref[t], 0, 0)),
            pl.BlockSpec((1, HID, H), lambda t, texp_ref: (texp_ref[t], 0, 0)),
            pl.BlockSpec((1, 1, H), lambda t, texp_ref: (texp_ref[t], 0, 0)),
        ],
        out_specs=pl.BlockSpec((T, H), lambda t, texp_ref: (t, 0)),
    )
    Yg = pl.pallas_call(
        _make_sparse_experts(),
        grid_spec=grid_spec,
        out_shape=jax.ShapeDtypeStruct((Pmax, H), jnp.float32),
    )(texp, Xg, wbcast, Wab, ba3, Wbb, bb3)

    out = _make_combine(N, H, NW, NC, CT)(Yg, posab)
    return out.reshape(B, M, H)


# speedup vs baseline: 1.1610x; 1.1610x over previous
import functools

import jax
import jax.numpy as jnp
from jax import lax
from jax.experimental import pallas as pl
from jax.experimental.pallas import tpu as pltpu
from jax.experimental.pallas import tpu_sc as plsc

_INVALID = 1 << 20


def _make_router(N, HP, H, E, B, TT, EP):
    NT = N // TT
    tiles_per_batch = NT // B

    def body(x_ref, sp_ref, aq_ref, w1x_ref, w1q_ref, w1s_ref, w2_ref, b2_ref,
             w_ref, r_ref, cnt_ref, carry_ref):
        t = pl.program_id(0)
        bidx = t // tiles_per_batch
        hdn = jnp.dot(x_ref[...], w1x_ref[...], preferred_element_type=jnp.float32)
        abias = jnp.dot(aq_ref[...], w1q_ref[...], preferred_element_type=jnp.float32)
        rows = jax.lax.broadcasted_iota(jnp.int32, abias.shape, 0)
        brow = jnp.sum(jnp.where(rows == bidx, abias, 0.0), axis=0, keepdims=True)
        hdn = jnp.tanh(hdn + brow + sp_ref[:, 0:1] * w1s_ref[0:1, :])
        logits = jnp.dot(hdn, w2_ref[...], preferred_element_type=jnp.float32)
        logits = logits + b2_ref[0:1, :]
        col = jax.lax.broadcasted_iota(jnp.int32, logits.shape, 1)
        valid = col < E
        logits = jnp.where(valid, logits, -jnp.inf)
        m = jnp.max(logits, axis=1, keepdims=True)
        ex = jnp.where(valid, jnp.exp(logits - m), 0.0)
        gate = ex / jnp.sum(ex, axis=1, keepdims=True)
        m1 = jnp.max(gate, axis=1, keepdims=True)
        i1 = jnp.min(jnp.where(gate == m1, col, EP + 1), axis=1, keepdims=True)
        sel1 = col == i1
        g2 = jnp.where(sel1 | ~valid, -1.0, gate)
        m2 = jnp.max(g2, axis=1, keepdims=True)
        i2 = jnp.min(jnp.where(g2 == m2, col, EP + 1), axis=1, keepdims=True)
        sel = sel1 | (col == i2)
        w_ref[...] = jnp.where(sel, gate, 0.0)

        onehot = sel.astype(jnp.int32)
        incl = onehot
        sh = 1
        while sh < TT:
            incl = incl + jnp.concatenate(
                [jnp.zeros((sh, EP), jnp.int32), incl[:-sh]], axis=0)
            sh *= 2
        excl = incl - onehot

        @pl.when(t == 0)
        def _init():
            carry_ref[...] = jnp.zeros_like(carry_ref)

        carry = carry_ref[0:1, :]
        r_ref[...] = jnp.where(sel, carry + excl, _INVALID)
        newc = carry + incl[TT - 1:TT, :]
        carry_ref[...] = jnp.broadcast_to(newc, carry_ref.shape)
        cnt_ref[...] = jnp.broadcast_to(newc, cnt_ref.shape)

    return body


def _make_dispatch_gather(N, E, T, Pmax, NW, NC, H):
    ENT = N * E // NW
    NCH = ENT // 16
    NV = ENT // 4
    NVR = NV // 128
    TPT = N // NW
    CX = 32
    NCX = TPT // CX
    mesh = plsc.VectorSubcoreMesh(core_axis_name="c", subcore_axis_name="s")

    @functools.partial(
        pl.kernel, mesh=mesh,
        compiler_params=pltpu.CompilerParams(needs_layout_passes=False),
        out_type=[
            jax.ShapeDtypeStruct((Pmax + 16,), jnp.float32),
            jax.ShapeDtypeStruct((2 * N + 16,), jnp.int32),
            jax.ShapeDtypeStruct((Pmax, H), jnp.float32),
        ],
        scratch_types=[
            pltpu.VMEM((ENT,), jnp.int32),
            pltpu.VMEM((ENT,), jnp.float32),
            pltpu.VMEM((16,), jnp.int32),
            pltpu.VMEM((16,), jnp.int32),
            pltpu.VMEM((NVR, 128), jnp.int32),
            pltpu.VMEM((NVR, 128), jnp.int32),
            pltpu.VMEM((NVR, 128), jnp.float32),
            pltpu.VMEM((NCX, CX), jnp.int32),
            pltpu.VMEM((NCX, CX), jnp.int32),
            pltpu.VMEM((3, CX, H), jnp.float32),
            pltpu.SemaphoreType.DMA,
            pltpu.SemaphoreType.DMA,
            pltpu.SemaphoreType.DMA,
            pltpu.SemaphoreType.DMA,
            pltpu.SemaphoreType.DMA,
            pltpu.SemaphoreType.DMA,
            pltpu.SemaphoreType.DMA,
        ],
    )
    def k(rflat, wflat, cnt_hbm, x_hbm, ws_hbm, posab_hbm, xg_hbm,
          rv, wv, cv, basev, posl, slotl, wvl, pA, pB, xbuf,
          sem, sl0, sl1, sl2, ss0, ss1, ss2):
        wid = lax.axis_index("s") * NC + lax.axis_index("c")
        base_ent = wid * ENT
        tok0 = wid * TPT
        E3 = E.bit_length() - 1
        lsem = [sl0, sl1, sl2]
        ssem = [ss0, ss1, ss2]
        loads = [None] * NCX
        for cx in range(min(3, NCX)):
            loads[cx] = pltpu.async_copy(
                x_hbm.at[pl.ds(tok0 + cx * CX, CX)], xbuf.at[cx], lsem[cx])
        pltpu.sync_copy(rflat.at[pl.ds(base_ent, ENT)], rv)
        pltpu.sync_copy(wflat.at[pl.ds(base_ent, ENT)], wv)
        pltpu.sync_copy(cnt_hbm, cv)
        cvec = cv[...]
        pc = ((cvec + (T - 1)) // T) * T
        incl = plsc.cumsum(pc)
        basev[...] = incl - pc
        lane = lax.iota(jnp.int32, 16)
        e16 = jnp.bitwise_and(lane, E - 1)

        def chunk(ci, acc):
            off = ci * 16
            r16 = rv[pl.ds(off, 16)]
            w16 = wv[pl.ds(off, 16)]
            gidx = base_ent + off + lane
            i16 = lax.shift_right_logical(gidx, E3)
            be = plsc.load_gather(basev, [e16])
            pos = be + r16
            isok = r16 < _INVALID
            vi = jnp.where(isok, 1, 0).astype(jnp.int32)
            grp0 = jnp.where(lane < 8, vi, 0)
            grp1 = jnp.where(lane >= 8, vi, 0)
            rank = jnp.where(lane < 8, plsc.cumsum(grp0), plsc.cumsum(grp1))
            slot = 2 * i16 + rank - 1
            cidx = ci * 4 + jnp.where(lane < 8, 0, 2) + rank - 1
            crow = cidx >> 7
            ccol = jnp.bitwise_and(cidx, 127)
            plsc.store_scatter(posl, [crow, ccol], pos, mask=isok)
            plsc.store_scatter(slotl, [crow, ccol], slot, mask=isok)
            plsc.store_scatter(wvl, [crow, ccol], w16, mask=isok)
            lt = i16 - tok0
            xrow = lt >> 5
            xcol = jnp.bitwise_and(lt, CX - 1)
            plsc.store_scatter(pA, [xrow, xcol], pos, mask=isok & (rank == 1))
            plsc.store_scatter(pB, [xrow, xcol], pos, mask=isok & (rank == 2))
            return acc

        lax.fori_loop(0, NCH, chunk, 0)

        copies = []
        for j in range(NVR):
            copies.append(pltpu.async_copy(wvl.at[j], ws_hbm.at[posl.at[j]], sem))
            copies.append(pltpu.async_copy(posl.at[j], posab_hbm.at[slotl.at[j]], sem))

        prev_sc = [None, None, None]
        for cx in range(NCX):
            b = cx % 3
            loads[cx].wait()
            prev_sc[b] = [
                pltpu.async_copy(xbuf.at[b], xg_hbm.at[pA.at[cx]], ssem[b]),
                pltpu.async_copy(xbuf.at[b], xg_hbm.at[pB.at[cx]], ssem[b]),
            ]
            if cx + 3 < NCX:
                for c in prev_sc[b]:
                    c.wait()
                prev_sc[b] = None
                loads[cx + 3] = pltpu.async_copy(
                    x_hbm.at[pl.ds(tok0 + (cx + 3) * CX, CX)],
                    xbuf.at[b], lsem[b])
        for pend in prev_sc:
            if pend is not None:
                for c in pend:
                    c.wait()
        for c in copies:
            c.wait()

    return k


def _make_sparse_experts():
    def body(texp_ref, xg_ref, w_ref, wa_ref, ba_ref, wb_ref, bb_ref, yg_ref):
        h1 = jnp.dot(xg_ref[...], wa_ref[0], preferred_element_type=jnp.float32)
        h1 = jnp.maximum(h1 + ba_ref[0], 0.0)
        y = jnp.dot(h1, wb_ref[0], preferred_element_type=jnp.float32)
        yg_ref[...] = (y + bb_ref[0]) * w_ref[:, 0:1]

    return body


def _make_combine(N, H, NW, NC, CT):
    TPT = N // NW
    mesh = plsc.VectorSubcoreMesh(core_axis_name="c", subcore_axis_name="s")

    NCH6 = TPT // CT

    @functools.partial(
        pl.kernel, mesh=mesh,
        out_type=jax.ShapeDtypeStruct((N, H), jnp.float32),
        scratch_types=[
            pltpu.VMEM((2 * TPT,), jnp.int32),
            pltpu.VMEM((2, 2 * CT, H), jnp.float32),
            pltpu.VMEM((2, CT, H), jnp.float32),
            pltpu.SemaphoreType.DMA,
            pltpu.SemaphoreType.DMA,
            pltpu.SemaphoreType.DMA,
            pltpu.SemaphoreType.DMA,
        ],
    )
    def k(yg_hbm, posab_hbm, out_hbm, idxv, rows, obuf, g0, g1, w0, w1):
        wid = lax.axis_index("s") * NC + lax.axis_index("c")
        base = wid * TPT
        HC = H // 16
        pltpu.sync_copy(posab_hbm.at[pl.ds(2 * base, 2 * TPT)], idxv)
        gsem = [g0, g1]
        wsem = [w0, w1]
        gat = [None] * NCH6
        prev_wb = [None, None]
        gat[0] = pltpu.async_copy(
            yg_hbm.at[idxv.at[pl.ds(0, 2 * CT)]], rows.at[0], gsem[0])
        for ci in range(NCH6):
            b = ci & 1
            if ci + 1 < NCH6:
                gat[ci + 1] = pltpu.async_copy(
                    yg_hbm.at[idxv.at[pl.ds(2 * (ci + 1) * CT, 2 * CT)]],
                    rows.at[(ci + 1) & 1], gsem[(ci + 1) & 1])
            gat[ci].wait()
            if prev_wb[b] is not None:
                prev_wb[b].wait()
                prev_wb[b] = None

            def add(q, a2):
                j = q // HC
                s = (q % HC) * 16
                obuf[b, j, pl.ds(s, 16)] = (
                    rows[b, 2 * j, pl.ds(s, 16)] + rows[b, 2 * j + 1, pl.ds(s, 16)])
                return a2

            lax.fori_loop(0, CT * HC, add, 0)
            prev_wb[b] = pltpu.async_copy(
                obuf.at[b], out_hbm.at[pl.ds(base + ci * CT, CT)], wsem[b])
        for pend in prev_wb:
            if pend is not None:
                pend.wait()

    return k


def kernel(token_x, aspect_q, token_score, W1, b1, W2, b2, Wa, ba, Wb, bb):
    B, M, H = token_x.shape
    E = Wa.shape[0]
    HID = Wa.shape[2]
    N = B * M
    TT = 256
    EP = 128
    T = 256
    NT = N // TT
    HP = H + 128
    Pmax = 2 * N + E * T
    NTS = Pmax // T
    NTT = ((NTS + 15) // 16) * 16
    NW, NC = 32, 2
    CG = 64
    CT = 16

    X = token_x.reshape(N, H)
    ts = token_score.reshape(N, 1)
    sp = jnp.broadcast_to(ts, (N, 128))
    w1sp = jnp.broadcast_to(W1[2 * H][None, :], (8, H))
    aqp = jnp.zeros((8, HP), jnp.float32)
    aqp = aqp.at[:B, :H].set(aspect_q)
    aqp = aqp.at[:, H].set(1.0)
    W1q = jnp.concatenate(
        [W1[H:2 * H], b1[None, :], jnp.zeros((127, H), jnp.float32)], axis=0)
    W2p = jnp.pad(W2, ((0, 0), (0, EP - E)))
    b2p = jnp.broadcast_to(jnp.pad(b2, (0, EP - E))[None, :], (8, EP))

    w_full, r_full, cnts = pl.pallas_call(
        _make_router(N, HP, H, E, B, TT, EP),
        grid=(NT,),
        in_specs=[
            pl.BlockSpec((TT, H), lambda t: (t, 0)),
            pl.BlockSpec((TT, 128), lambda t: (t, 0)),
            pl.BlockSpec((8, HP), lambda t: (0, 0)),
            pl.BlockSpec((H, H), lambda t: (0, 0)),
            pl.BlockSpec((HP, H), lambda t: (0, 0)),
            pl.BlockSpec((8, H), lambda t: (0, 0)),
            pl.BlockSpec((H, EP), lambda t: (0, 0)),
            pl.BlockSpec((8, EP), lambda t: (0, 0)),
        ],
        out_specs=[
            pl.BlockSpec((TT, EP), lambda t: (t, 0)),
            pl.BlockSpec((TT, EP), lambda t: (t, 0)),
            pl.BlockSpec((8, EP), lambda t: (0, 0)),
        ],
        out_shape=[
            jax.ShapeDtypeStruct((N, EP), jnp.float32),
            jax.ShapeDtypeStruct((N, EP), jnp.int32),
            jax.ShapeDtypeStruct((8, EP), jnp.int32),
        ],
        scratch_shapes=[pltpu.VMEM((8, EP), jnp.int32)],
    )(X, sp, aqp, W1[:H], W1q, w1sp, W2p, b2p)

    rflat = r_full[:, :E].reshape(-1)
    wflat = w_full[:, :E].reshape(-1)
    cnt16 = jnp.zeros((16,), jnp.int32).at[:E].set(cnts[0, :E])

    ws, posab, Xg = _make_dispatch_gather(N, E, T, Pmax, NW, NC, H)(
        rflat, wflat, cnt16, X)

    ends = jnp.cumsum(((cnts[0, :E] + T - 1) // T) * T)
    tile_base = jnp.arange(NTS, dtype=jnp.int32) * T
    texp = jnp.minimum(
        jnp.sum((tile_base[:, None] >= ends[None, :]).astype(jnp.int32), axis=1),
        E - 1)

    wbcast = jnp.broadcast_to(ws[:Pmax, None], (Pmax, EP))
    ba3 = ba.reshape(E, 1, HID)
    bb3 = bb.reshape(E, 1, H)
    grid_spec = pltpu.PrefetchScalarGridSpec(
        num_scalar_prefetch=1,
        grid=(NTS,),
        in_specs=[
            pl.BlockSpec((T, H), lambda t, texp_ref: (t, 0)),
            pl.BlockSpec((T, EP), lambda t, texp_ref: (t, 0)),
            pl.BlockSpec((1, H, HID), lambda t, texp_ref: (texp_ref[t], 0, 0)),
            pl.BlockSpec((1, 1, HID), lambda t, texp_ref: (texp_ref[t], 0, 0)),
            pl.BlockSpec((1, HID, H), lambda t, texp_ref: (texp_ref[t], 0, 0)),
            pl.BlockSpec((1, 1, H), lambda t, texp_ref: (texp_ref[t], 0, 0)),
        ],
        out_specs=pl.BlockSpec((T, H), lambda t, texp_ref: (t, 0)),
    )
    Yg = pl.pallas_call(
        _make_sparse_experts(),
        grid_spec=grid_spec,
        out_shape=jax.ShapeDtypeStruct((Pmax, H), jnp.float32),
    )(texp, Xg, wbcast, Wa, ba3, Wb, bb3)

    out = _make_combine(N, H, NW, NC, CT)(Yg, posab)
    return out.reshape(B, M, H)
